# Initial kernel scaffold; baseline (speedup 1.0000x reference)
import jax
import jax.numpy as jnp
from jax.experimental import pallas as pl


def kernel(input_ids, weight):
    raise NotImplementedError("write your pallas kernel here")


# speedup vs baseline: 1.7732x; 1.7732x over previous
import functools

import jax
import jax.numpy as jnp
from jax import lax
from jax.experimental import pallas as pl
from jax.experimental.pallas import tpu as pltpu
from jax.experimental.pallas import tpu_sc as plsc

_NUM_CORES = 2
_NUM_SUBCORES = 16
_NW = _NUM_CORES * _NUM_SUBCORES
_CHUNK = 32


@functools.lru_cache(maxsize=None)
def _build_gather(B, D):
    b_per_w = B // _NW
    n_chunks = b_per_w // _CHUNK
    mesh = plsc.VectorSubcoreMesh(core_axis_name="c", subcore_axis_name="s")

    @functools.partial(
        pl.kernel,
        mesh=mesh,
        out_type=jax.ShapeDtypeStruct((B, D), jnp.float32),
        scratch_types=[
            pltpu.VMEM((n_chunks, _CHUNK), jnp.int32),
            pltpu.VMEM((_CHUNK, D), jnp.float32),
            pltpu.VMEM((_CHUNK, D), jnp.float32),
            pltpu.SemaphoreType.DMA,
            pltpu.SemaphoreType.DMA,
        ],
    )
    def gather_kernel(table_hbm, idx_hbm, out_hbm, idx_v, buf0, buf1, sem0, sem1):
        wid = lax.axis_index("s") * _NUM_CORES + lax.axis_index("c")
        base = wid * b_per_w
        pltpu.sync_copy(idx_hbm.at[wid], idx_v)
        bufs = (buf0, buf1)
        sems = (sem0, sem1)
        for b in range(2):
            pltpu.async_copy(table_hbm.at[idx_v.at[b]], bufs[b], sems[b])

        def step(g, carry):
            for b in range(2):
                chunk = 2 * g + b
                pltpu.make_async_copy(
                    table_hbm.at[idx_v.at[chunk]], bufs[b], sems[b]
                ).wait()
                pltpu.sync_copy(bufs[b], out_hbm.at[pl.ds(base + chunk * _CHUNK, _CHUNK)])
                pltpu.async_copy(table_hbm.at[idx_v.at[chunk + 2]], bufs[b], sems[b])
            return carry

        lax.fori_loop(0, n_chunks // 2 - 1, step, 0)
        for b in range(2):
            chunk = n_chunks - 2 + b
            pltpu.make_async_copy(
                table_hbm.at[idx_v.at[chunk]], bufs[b], sems[b]
            ).wait()
            pltpu.sync_copy(bufs[b], out_hbm.at[pl.ds(base + chunk * _CHUNK, _CHUNK)])

    return gather_kernel


def kernel(input_ids, weight):
    orig_shape = input_ids.shape
    D = weight.shape[1]
    B = input_ids.size
    idx = input_ids.reshape(_NW, (B // _NW) // _CHUNK, _CHUNK).astype(jnp.int32)
    out = _build_gather(B, D)(weight.astype(jnp.float32), idx)
    return out.reshape(*orig_shape, D)
